# Initial kernel scaffold; baseline (speedup 1.0000x reference)
import jax
import jax.numpy as jnp
from jax.experimental import pallas as pl


def kernel(nodes, grid, edge_index, edge_attr, batch_size, image_size, proj_w1, proj_b1, proj_w2, proj_b2, gcn_w, gcn_b, ln_g, ln_b, dec_w1, dec_b1, dec_w2, dec_b2):
    raise NotImplementedError("write your pallas kernel here")


# speedup vs baseline: 8.0969x; 8.0969x over previous
import functools

import jax
import jax.numpy as jnp
from jax import lax
from jax.experimental import pallas as pl
from jax.experimental.pallas import tpu as pltpu
from jax.experimental.pallas import tpu_sc as plsc

_N = 10000
_E = 320000
_D = 128
_DEPTH = 4

_NC = 2
_NS = 16
_NW = _NC * _NS
_CHUNK = 128
_NCHUNKS = 79
_EW = _CHUNK * _NCHUNKS
_EP = _EW * _NW
_ACC_N = 10240
_RPS = _ACC_N // _NS
_TRASH = 10016

_mesh = plsc.VectorSubcoreMesh(core_axis_name="c", subcore_axis_name="s")


def _sc_agg(hs, srcp, dstp, zeros):
  @functools.partial(
      pl.kernel,
      out_type=jax.ShapeDtypeStruct((_NC * _ACC_N, _D), jnp.float32),
      mesh=_mesh,
      scratch_types=[
          pltpu.VMEM_SHARED((_ACC_N, _D), jnp.float32),
          pltpu.VMEM((_CHUNK,), jnp.int32),
          pltpu.VMEM((_CHUNK,), jnp.int32),
          pltpu.VMEM((_CHUNK, _D), jnp.float32),
          pltpu.SemaphoreType.DMA,
      ],
  )
  def k(hs_hbm, src_hbm, dst_hbm, zeros_hbm, out_hbm, acc, src_v, dst_v,
        rows_v, sem):
    c = lax.axis_index("c")
    s = lax.axis_index("s")
    row0 = s * _RPS
    pltpu.sync_copy(zeros_hbm.at[pl.ds(row0, _RPS)], acc.at[pl.ds(row0, _RPS)])
    plsc.subcore_barrier()
    base = c * (_NS * _EW) + s * _EW

    @pl.loop(0, _NCHUNKS)
    def _(t):
      off = base + t * _CHUNK
      pltpu.sync_copy(src_hbm.at[pl.ds(off, _CHUNK)], src_v)
      pltpu.sync_copy(dst_hbm.at[pl.ds(off, _CHUNK)], dst_v)
      pltpu.async_copy(hs_hbm.at[src_v], rows_v, sem).wait()
      pltpu.sync_copy(rows_v, acc.at[dst_v], add=True)

    plsc.subcore_barrier()
    pltpu.sync_copy(acc.at[pl.ds(row0, _RPS)],
                    out_hbm.at[pl.ds(c * _ACC_N + row0, _RPS)])

  return k(hs, srcp, dstp, zeros)


def _sc_deg(dstp, ones, zeros):
  @functools.partial(
      pl.kernel,
      out_type=jax.ShapeDtypeStruct((_NC * _ACC_N, _D), jnp.float32),
      mesh=_mesh,
      scratch_types=[
          pltpu.VMEM_SHARED((_ACC_N, _D), jnp.float32),
          pltpu.VMEM((_CHUNK,), jnp.int32),
          pltpu.VMEM((_CHUNK, _D), jnp.float32),
          pltpu.SemaphoreType.DMA,
      ],
  )
  def k(dst_hbm, ones_hbm, zeros_hbm, out_hbm, acc, dst_v, ones_v, sem):
    c = lax.axis_index("c")
    s = lax.axis_index("s")
    row0 = s * _RPS
    pltpu.sync_copy(zeros_hbm.at[pl.ds(row0, _RPS)], acc.at[pl.ds(row0, _RPS)])
    pltpu.sync_copy(ones_hbm, ones_v)
    plsc.subcore_barrier()
    base = c * (_NS * _EW) + s * _EW

    @pl.loop(0, _NCHUNKS)
    def _(t):
      off = base + t * _CHUNK
      pltpu.sync_copy(dst_hbm.at[pl.ds(off, _CHUNK)], dst_v)
      pltpu.sync_copy(ones_v, acc.at[dst_v], add=True)

    plsc.subcore_barrier()
    pltpu.sync_copy(acc.at[pl.ds(row0, _RPS)],
                    out_hbm.at[pl.ds(c * _ACC_N + row0, _RPS)])

  return k(dstp, ones, zeros)


def _proj_body(x_ref, w1_ref, b1_ref, w2_ref, b2_ref, o_ref):
  h = jnp.dot(x_ref[...], w1_ref[...], preferred_element_type=jnp.float32)
  h = jnp.maximum(h + b1_ref[...], 0.0)
  y = jnp.dot(h, w2_ref[...], preferred_element_type=jnp.float32)
  o_ref[...] = jax.nn.gelu(y + b2_ref[...])


def _dinv_body(deg_ref, o_ref):
  d0 = deg_ref[0:_N, 0:1]
  d1 = deg_ref[_ACC_N:_ACC_N + _N, 0:1]
  o_ref[...] = lax.rsqrt(d0 + d1 + 1.0)


def _pre_body(x_ref, w_ref, dinv_ref, o_ref):
  h = jnp.dot(x_ref[...], w_ref[...], preferred_element_type=jnp.float32)
  o_ref[...] = h * dinv_ref[...]


def _post_body(parts_ref, hs_ref, dinv_ref, b_ref, g_ref, bb_ref, o_ref,
               *, last):
  agg = parts_ref[0:_N, :] + parts_ref[_ACC_N:_ACC_N + _N, :]
  y = dinv_ref[...] * (agg + hs_ref[...]) + b_ref[...]
  if last:
    o_ref[...] = y
  else:
    y = jax.nn.gelu(y)
    mu = jnp.mean(y, axis=-1, keepdims=True)
    yc = y - mu
    var = jnp.mean(yc * yc, axis=-1, keepdims=True)
    o_ref[...] = yc * lax.rsqrt(var + 1e-5) * g_ref[...] + bb_ref[...]


def _dec_body(x_ref, w1_ref, b1_ref, w2_ref, b2_ref, o_ref):
  h = jnp.dot(x_ref[...], w1_ref[...], preferred_element_type=jnp.float32)
  h = jnp.maximum(h + b1_ref[...], 0.0)
  y = jnp.dot(h, w2_ref[...], preferred_element_type=jnp.float32)
  o_ref[...] = y + b2_ref[...]


def _tc(body, n_out_cols, *args):
  return pl.pallas_call(
      body,
      out_shape=jax.ShapeDtypeStruct((_N, n_out_cols), jnp.float32),
  )(*args)


def kernel(nodes, grid, edge_index, edge_attr, batch_size, image_size,
           proj_w1, proj_b1, proj_w2, proj_b2, gcn_w, gcn_b, ln_g, ln_b,
           dec_w1, dec_b1, dec_w2, dec_b2):
  del edge_attr, batch_size, image_size

  x0 = jnp.concatenate([nodes, grid], axis=-1)
  x0 = jnp.pad(x0, ((0, 0), (0, 16 - x0.shape[1])))
  w1p = jnp.pad(proj_w1, ((0, 16 - proj_w1.shape[0]), (0, 0)))

  src = edge_index[0]
  dst = edge_index[1]
  npad = _EP - _E
  srcp = jnp.concatenate([src, jnp.zeros((npad,), jnp.int32)])
  dstp = jnp.concatenate([dst, jnp.full((npad,), _TRASH, jnp.int32)])

  zeros = jnp.zeros((_ACC_N, _D), jnp.float32)
  ones = jnp.ones((_CHUNK, _D), jnp.float32)

  b1 = proj_b1.reshape(1, -1)
  b2 = proj_b2.reshape(1, -1)
  g = ln_g.reshape(1, -1)
  bb = ln_b.reshape(1, -1)
  db1 = dec_b1.reshape(1, -1)
  db2 = dec_b2.reshape(1, -1)

  deg_parts = _sc_deg(dstp, ones, zeros)
  x = _tc(_proj_body, _D, x0, w1p, b1, proj_w2, b2)
  dinv = _tc(_dinv_body, 1, deg_parts)

  for i in range(_DEPTH):
    hs = _tc(_pre_body, _D, x, gcn_w[i], dinv)
    parts = _sc_agg(hs, srcp, dstp, zeros)
    x = _tc(
        functools.partial(_post_body, last=(i == _DEPTH - 1)),
        _D, parts, hs, dinv, gcn_b[i].reshape(1, -1), g, bb)

  return _tc(_dec_body, 1, x, dec_w1, db1, dec_w2, db2)
